# Initial kernel scaffold; baseline (speedup 1.0000x reference)
import jax
import jax.numpy as jnp
from jax.experimental import pallas as pl


def kernel(x, conv_w, gamma, beta, running_mean, running_var):
    raise NotImplementedError("write your pallas kernel here")


# speedup vs baseline: 3.4887x; 3.4887x over previous
import jax
import jax.numpy as jnp
from jax.experimental import pallas as pl
from jax.experimental.pallas import tpu as pltpu


def _stem_kernel(x_ref, w_ref, b_ref, o_ref):
    cout = o_ref.shape[1]
    hp, wp = o_ref.shape[2], o_ref.shape[3]
    ho, wc = 2 * hp, x_ref.shape[3] - 3

    xb = x_ref[0]

    cols = []
    for b in range(4):
        xcb = xb[:, :, b:b + wc]
        for a in range(4):
            cols.append(xcb[:, a:a + ho, :])
    p = jnp.concatenate(cols, axis=0).reshape(16 * xb.shape[0], ho * wc)

    acc = jnp.dot(w_ref[...], p, preferred_element_type=jnp.float32)
    acc = jnp.maximum(acc + b_ref[...], 0.0)
    y = acc.reshape(cout, ho, wc)

    yr = y.reshape(cout, hp, 2, wc)
    ye = yr[:, :, 0, :]
    yo = yr[:, :, 1, :]
    zrow = jnp.zeros((cout, 1, wc), jnp.float32)
    yo_prev = jnp.concatenate([zrow, yo[:, :-1, :]], axis=1)
    rsub = jnp.maximum(ye, jnp.maximum(yo, yo_prev))

    zcol = jnp.zeros((cout, hp, 1), jnp.float32)
    lf = jnp.concatenate([zcol, rsub[:, :, :-1]], axis=2)
    rt = jnp.concatenate([rsub[:, :, 1:], zcol], axis=2)
    cmax = jnp.maximum(rsub, jnp.maximum(lf, rt))

    ii = jax.lax.broadcasted_iota(jnp.int32, (wc, wc), 0)
    jj = jax.lax.broadcasted_iota(jnp.int32, (wc, wc), 1)
    sel = (ii == 2 * jj).astype(jnp.float32)
    res = jnp.dot(cmax.reshape(cout * hp, wc), sel,
                  preferred_element_type=jnp.float32)
    o_ref[0] = res.reshape(cout, hp, wc)[:, :, :wp]


def kernel(x, conv_w, gamma, beta, running_mean, running_var):
    eps = 1e-5
    B, Cin, H, W = x.shape
    Cout, _, KH, KW = conv_w.shape
    Ho, Wo = H // 2, W // 2
    Hp, Wp = Ho // 2, Wo // 2
    Wc = -(-(Wo + 1) // 128) * 128

    scale = gamma * jax.lax.rsqrt(running_var + eps)
    shift = beta - running_mean * scale

    w8 = jnp.pad(conv_w, ((0, 0), (0, 0), (0, 1), (0, 1)))
    w8 = w8.reshape(Cout, Cin, 4, 2, 4, 2)
    w2 = w8.transpose(0, 4, 2, 1, 3, 5).reshape(Cout, Cin * 64)
    w2 = (w2 * scale[:, None]).astype(jnp.bfloat16)
    bias = shift.astype(jnp.float32).reshape(Cout, 1)

    Hs, Ws = Ho + 3, Wc + 3
    x_pad = jnp.pad(x, ((0, 0), (0, 0), (3, 2 * Hs - H - 3),
                        (3, 2 * Ws - W - 3)))
    x_s2d = (x_pad.reshape(B, Cin, Hs, 2, Ws, 2)
             .transpose(0, 1, 3, 5, 2, 4)
             .reshape(B, 4 * Cin, Hs, Ws)
             .astype(jnp.bfloat16))

    out = pl.pallas_call(
        _stem_kernel,
        out_shape=jax.ShapeDtypeStruct((B, Cout, Hp, Wp), x.dtype),
        grid_spec=pltpu.PrefetchScalarGridSpec(
            num_scalar_prefetch=0,
            grid=(B,),
            in_specs=[
                pl.BlockSpec((1, 4 * Cin, Hs, Ws), lambda b: (b, 0, 0, 0)),
                pl.BlockSpec((Cout, Cin * 64), lambda b: (0, 0)),
                pl.BlockSpec((Cout, 1), lambda b: (0, 0)),
            ],
            out_specs=pl.BlockSpec((1, Cout, Hp, Wp), lambda b: (b, 0, 0, 0)),
        ),
        compiler_params=pltpu.CompilerParams(
            dimension_semantics=("parallel",),
        ),
    )(x_s2d, w2, bias)
    return out


# speedup vs baseline: 20.8215x; 5.9683x over previous
import jax
import jax.numpy as jnp
from jax.experimental import pallas as pl
from jax.experimental.pallas import tpu as pltpu


def _stem_kernel(x_ref, w_ref, b_ref, o_ref, p_ref):
    cin, h, w = x_ref.shape[1], x_ref.shape[2], x_ref.shape[3]
    cout = o_ref.shape[1]
    hp, wp = o_ref.shape[2], o_ref.shape[3]
    hs = 2 * hp + 3
    wc = p_ref.shape[1] // (2 * hp)
    wsc = wc + 3

    xb = x_ref[0]
    ztop = jnp.zeros((cin, 3, w), jnp.float32)
    zbot = jnp.zeros((cin, 2 * hs - h - 3, w), jnp.float32)
    xr = jnp.concatenate([ztop, xb, zbot], axis=1).astype(jnp.bfloat16)
    flat = xr.reshape(cin * 2 * hs, w)

    ii = jax.lax.broadcasted_iota(jnp.int32, (w, wsc), 0)
    jj = jax.lax.broadcasted_iota(jnp.int32, (w, wsc), 1)
    sel0 = (ii == 2 * jj - 3).astype(jnp.bfloat16)
    sel1 = (ii == 2 * jj - 2).astype(jnp.bfloat16)
    xp0 = jnp.dot(flat, sel0, preferred_element_type=jnp.float32)
    xp1 = jnp.dot(flat, sel1, preferred_element_type=jnp.float32)
    xp0 = xp0.astype(jnp.bfloat16).reshape(cin, hs, 2, wsc)
    xp1 = xp1.astype(jnp.bfloat16).reshape(cin, hs, 2, wsc)
    xs = jnp.concatenate([xp0[:, :, 0, :], xp1[:, :, 0, :],
                          xp0[:, :, 1, :], xp1[:, :, 1, :]], axis=0)

    nch = 4 * cin
    for b in range(4):
        xcb = xs[:, :, b:b + wc]
        for a in range(4):
            blk = xcb[:, a:a + 2 * hp, :]
            p_ref[(b * 4 + a) * nch:(b * 4 + a + 1) * nch, :] = (
                blk.reshape(nch, 2 * hp * wc))

    acc = jnp.dot(w_ref[...], p_ref[...], preferred_element_type=jnp.float32)
    acc = jnp.maximum(acc + b_ref[...], 0.0)
    y = acc.reshape(cout, 2 * hp, wc)

    yr = y.reshape(cout, hp, 2, wc)
    ye = yr[:, :, 0, :]
    yo = yr[:, :, 1, :]
    zrow = jnp.zeros((cout, 1, wc), jnp.float32)
    yo_prev = jnp.concatenate([zrow, yo[:, :-1, :]], axis=1)
    rsub = jnp.maximum(ye, jnp.maximum(yo, yo_prev))

    zcol = jnp.zeros((cout, hp, 1), jnp.float32)
    lf = jnp.concatenate([zcol, rsub[:, :, :-1]], axis=2)
    rt = jnp.concatenate([rsub[:, :, 1:], zcol], axis=2)
    cmax = jnp.maximum(rsub, jnp.maximum(lf, rt))

    oi = jax.lax.broadcasted_iota(jnp.int32, (wc, wc), 0)
    oj = jax.lax.broadcasted_iota(jnp.int32, (wc, wc), 1)
    sel = (oi == 2 * oj).astype(jnp.float32)
    res = jnp.dot(cmax.reshape(cout * hp, wc), sel,
                  preferred_element_type=jnp.float32)
    o_ref[0] = res.reshape(cout, hp, wc)[:, :, :wp]


def kernel(x, conv_w, gamma, beta, running_mean, running_var):
    eps = 1e-5
    B, Cin, H, W = x.shape
    Cout, _, KH, KW = conv_w.shape
    Ho, Wo = H // 2, W // 2
    Hp, Wp = Ho // 2, Wo // 2
    Wc = -(-(Wo + 1) // 128) * 128

    scale = gamma * jax.lax.rsqrt(running_var + eps)
    shift = beta - running_mean * scale

    w8 = jnp.pad(conv_w, ((0, 0), (0, 0), (0, 1), (0, 1)))
    w8 = w8.reshape(Cout, Cin, 4, 2, 4, 2)
    w2 = w8.transpose(0, 4, 2, 3, 5, 1).reshape(Cout, Cin * 64)
    w2 = (w2 * scale[:, None]).astype(jnp.bfloat16)
    bias = shift.astype(jnp.float32).reshape(Cout, 1)

    out = pl.pallas_call(
        _stem_kernel,
        out_shape=jax.ShapeDtypeStruct((B, Cout, Hp, Wp), x.dtype),
        grid_spec=pltpu.PrefetchScalarGridSpec(
            num_scalar_prefetch=0,
            grid=(B,),
            in_specs=[
                pl.BlockSpec((1, Cin, H, W), lambda b: (b, 0, 0, 0)),
                pl.BlockSpec((Cout, Cin * 64), lambda b: (0, 0)),
                pl.BlockSpec((Cout, 1), lambda b: (0, 0)),
            ],
            out_specs=pl.BlockSpec((1, Cout, Hp, Wp), lambda b: (b, 0, 0, 0)),
            scratch_shapes=[pltpu.VMEM((Cin * 64, Ho * Wc), jnp.bfloat16)],
        ),
        compiler_params=pltpu.CompilerParams(
            dimension_semantics=("parallel",),
        ),
    )(x, w2, bias)
    return out
